# baseline (speedup 1.0000x reference)
import functools

import jax
import jax.numpy as jnp
from jax import lax
from jax.experimental import pallas as pl
from jax.experimental.pallas import tpu as pltpu
from jax.experimental.pallas import tpu_sc as plsc

NUM_CORES = 1
NUM_SUBCORES = 16
NUM_TILES = NUM_CORES * NUM_SUBCORES
CHUNK = 512
SUBCHUNKS = 4
STAGE = CHUNK * SUBCHUNKS
LANES = 16


def _make_sc_call(N: int, nnz_pad: int):
    per_tile = nnz_pad // NUM_TILES
    n_stage = per_tile // STAGE
    assert n_stage % 2 == 0
    rows_per_tile = N // NUM_SUBCORES

    mesh = plsc.VectorSubcoreMesh(
        core_axis_name="c", subcore_axis_name="s", num_cores=NUM_CORES
    )

    @functools.partial(
        pl.kernel,
        out_type=jax.ShapeDtypeStruct((NUM_CORES, N, LANES), jnp.float32),
        mesh=mesh,
        scratch_types=dict(
            acc=pltpu.MemorySpace.VMEM_SHARED((N, LANES), jnp.float32),
            xsh=pltpu.MemorySpace.VMEM_SHARED((N, LANES), jnp.float32),
            cidx=pltpu.MemorySpace.VMEM((2, SUBCHUNKS, CHUNK), jnp.int32),
            ridx=pltpu.MemorySpace.VMEM((2, SUBCHUNKS, CHUNK), jnp.int32),
            vals=pltpu.MemorySpace.VMEM((2 * STAGE,), jnp.float32),
            g0=pltpu.MemorySpace.VMEM((CHUNK, LANES), jnp.float32),
            g1=pltpu.MemorySpace.VMEM((CHUNK, LANES), jnp.float32),
            g2=pltpu.MemorySpace.VMEM((CHUNK, LANES), jnp.float32),
            g3=pltpu.MemorySpace.VMEM((CHUNK, LANES), jnp.float32),
            g4=pltpu.MemorySpace.VMEM((CHUNK, LANES), jnp.float32),
            g5=pltpu.MemorySpace.VMEM((CHUNK, LANES), jnp.float32),
            g6=pltpu.MemorySpace.VMEM((CHUNK, LANES), jnp.float32),
            g7=pltpu.MemorySpace.VMEM((CHUNK, LANES), jnp.float32),
            lsem0=pltpu.SemaphoreType.DMA,
            lsem1=pltpu.SemaphoreType.DMA,
            gsem0=pltpu.SemaphoreType.DMA,
            gsem1=pltpu.SemaphoreType.DMA,
            gsem2=pltpu.SemaphoreType.DMA,
            gsem3=pltpu.SemaphoreType.DMA,
            ssem0=pltpu.SemaphoreType.DMA,
            ssem1=pltpu.SemaphoreType.DMA,
            ssem2=pltpu.SemaphoreType.DMA,
            ssem3=pltpu.SemaphoreType.DMA,
            gsem4=pltpu.SemaphoreType.DMA,
            gsem5=pltpu.SemaphoreType.DMA,
            gsem6=pltpu.SemaphoreType.DMA,
            gsem7=pltpu.SemaphoreType.DMA,
            ssem4=pltpu.SemaphoreType.DMA,
            ssem5=pltpu.SemaphoreType.DMA,
            ssem6=pltpu.SemaphoreType.DMA,
            ssem7=pltpu.SemaphoreType.DMA,
        ),
        compiler_params=pltpu.CompilerParams(use_tc_tiling_on_sc=False),
    )
    def sc_call(x16, rows_h, cols_h, vals_h, out_h, acc, xsh, cidx, ridx, vals,
                g0, g1, g2, g3, g4, g5, g6, g7, lsem0, lsem1,
                gsem0, gsem1, gsem2, gsem3, ssem0, ssem1, ssem2, ssem3,
                gsem4, gsem5, gsem6, gsem7, ssem4, ssem5, ssem6, ssem7):
        c = lax.axis_index("c")
        s = lax.axis_index("s")

        z16 = jnp.zeros((LANES,), jnp.float32)

        for i in range(CHUNK):
            g0[i] = z16
        for i in range(rows_per_tile // CHUNK):
            pltpu.sync_copy(
                g0, acc.at[pl.ds(s * rows_per_tile + i * CHUNK, CHUNK)]
            )
        xsl = pl.ds(s * rows_per_tile, rows_per_tile)
        pltpu.sync_copy(x16.at[xsl], xsh.at[xsl])
        plsc.subcore_barrier()

        tile_id = s * NUM_CORES + c
        cbase = tile_id * (per_tile // CHUNK)

        gbufs = (g0, g1, g2, g3, g4, g5, g6, g7)
        gsems = (gsem0, gsem1, gsem2, gsem3, gsem4, gsem5, gsem6, gsem7)
        ssems = (ssem0, ssem1, ssem2, ssem3, ssem4, ssem5, ssem6, ssem7)
        lsems = (lsem0, lsem1)

        def issue_linear(p, t):
            coff = cbase + t * SUBCHUNKS
            pltpu.async_copy(
                rows_h.at[pl.ds(coff, SUBCHUNKS)], ridx.at[p], lsems[p]
            )
            pltpu.async_copy(
                cols_h.at[pl.ds(coff, SUBCHUNKS)], cidx.at[p], lsems[p]
            )
            pltpu.async_copy(
                vals_h.at[pl.ds(coff * CHUNK, STAGE)],
                vals.at[pl.ds(p * STAGE, STAGE)],
                lsems[p],
            )

        def wait_linear(p):
            pltpu.make_async_copy(
                rows_h.at[pl.ds(0, SUBCHUNKS)], ridx.at[p], lsems[p]
            ).wait()
            pltpu.make_async_copy(
                cols_h.at[pl.ds(0, SUBCHUNKS)], cidx.at[p], lsems[p]
            ).wait()
            pltpu.make_async_copy(
                vals_h.at[pl.ds(0, STAGE)],
                vals.at[pl.ds(p * STAGE, STAGE)],
                lsems[p],
            ).wait()

        def compute_inplace(gb, voff):
            def group(i16, _):
                vv = vals[pl.ds(voff + i16 * 16, 16)]
                base = i16 * 16
                for u in range(16):
                    gb[base + u] = gb[base + u] * vv[u]
                return 0

            lax.fori_loop(0, CHUNK // 16, group, 0)

        def process_stage(p):
            ci = cidx.at[p]
            ri = ridx.at[p]
            gd = [
                pltpu.async_copy(xsh.at[ci.at[j]], gbufs[j], gsems[j])
                for j in range(SUBCHUNKS)
            ]
            sd = []
            for j in range(SUBCHUNKS):
                gd[j].wait()
                compute_inplace(gbufs[j], p * STAGE + j * CHUNK)
                sd.append(
                    pltpu.async_copy(
                        gbufs[j], acc.at[ri.at[j]], ssems[j], add=True
                    )
                )
            for d in sd:
                d.wait()

        issue_linear(0, 0)

        def pair_body(t2, _):
            t0 = 2 * t2
            wait_linear(0)
            issue_linear(1, t0 + 1)
            process_stage(0)
            wait_linear(1)

            @pl.when(t2 + 1 < n_stage // 2)
            def _():
                issue_linear(0, t0 + 2)

            process_stage(1)
            return 0

        lax.fori_loop(0, n_stage // 2, pair_body, 0)

        plsc.subcore_barrier()
        def out_body(i, _):
            sl = pl.ds(s * rows_per_tile + i * CHUNK, CHUNK)
            pltpu.sync_copy(acc.at[sl], g0)
            pltpu.sync_copy(g0, out_h.at[c].at[sl])
            return 0

        lax.fori_loop(0, rows_per_tile // CHUNK, out_body, 0)

    return sc_call


def _combine(partials):
    two, n, lanes = partials.shape

    def body(p_ref, o_ref):
        o_ref[...] = p_ref[0] + p_ref[1]

    return pl.pallas_call(
        body,
        out_shape=jax.ShapeDtypeStruct((n, lanes), jnp.float32),
    )(partials)


def kernel(x_batched, rows, cols, values):
    B, N = x_batched.shape
    nnz = rows.shape[0]

    unit = NUM_TILES * STAGE * 2
    nnz_pad = ((nnz + unit - 1) // unit) * unit
    pad = nnz_pad - nnz
    if pad:
        spread = (jnp.arange(pad, dtype=jnp.int32) * 97) % N
        rows = jnp.concatenate([rows, spread])
        cols = jnp.concatenate([cols, spread])
        values = jnp.concatenate([values, jnp.zeros((pad,), values.dtype)])

    x16 = jnp.concatenate(
        [x_batched.T, jnp.zeros((N, LANES - B), jnp.float32)], axis=1
    )

    rows2d = rows.reshape(-1, CHUNK)
    cols2d = cols.reshape(-1, CHUNK)
    out2 = _make_sc_call(N, nnz_pad)(x16, rows2d, cols2d, values)
    out = _combine(out2)
    return out[:, :B].T


# speedup vs baseline: 1.6303x; 1.6303x over previous
import functools

import jax
import jax.numpy as jnp
from jax import lax
from jax.experimental import pallas as pl
from jax.experimental.pallas import tpu as pltpu
from jax.experimental.pallas import tpu_sc as plsc

NUM_CORES = 2
NUM_SUBCORES = 16
NUM_TILES = NUM_CORES * NUM_SUBCORES
CHUNK = 512
SUBCHUNKS = 4
STAGE = CHUNK * SUBCHUNKS
LANES = 16


def _make_sc_call(N: int, nnz_pad: int):
    per_tile = nnz_pad // NUM_TILES
    n_stage = per_tile // STAGE
    assert n_stage % 2 == 0
    rows_per_tile = N // NUM_SUBCORES

    mesh = plsc.VectorSubcoreMesh(
        core_axis_name="c", subcore_axis_name="s", num_cores=NUM_CORES
    )

    @functools.partial(
        pl.kernel,
        out_type=jax.ShapeDtypeStruct((NUM_CORES, N, LANES), jnp.float32),
        mesh=mesh,
        scratch_types=dict(
            acc=pltpu.MemorySpace.VMEM_SHARED((N, LANES), jnp.float32),
            xsh=pltpu.MemorySpace.VMEM_SHARED((N, LANES), jnp.float32),
            cidx=pltpu.MemorySpace.VMEM((2, SUBCHUNKS, CHUNK), jnp.int32),
            ridx=pltpu.MemorySpace.VMEM((2, SUBCHUNKS, CHUNK), jnp.int32),
            vals=pltpu.MemorySpace.VMEM((2 * STAGE,), jnp.float32),
            g0=pltpu.MemorySpace.VMEM((CHUNK, LANES), jnp.float32),
            g1=pltpu.MemorySpace.VMEM((CHUNK, LANES), jnp.float32),
            g2=pltpu.MemorySpace.VMEM((CHUNK, LANES), jnp.float32),
            g3=pltpu.MemorySpace.VMEM((CHUNK, LANES), jnp.float32),
            g4=pltpu.MemorySpace.VMEM((CHUNK, LANES), jnp.float32),
            g5=pltpu.MemorySpace.VMEM((CHUNK, LANES), jnp.float32),
            g6=pltpu.MemorySpace.VMEM((CHUNK, LANES), jnp.float32),
            g7=pltpu.MemorySpace.VMEM((CHUNK, LANES), jnp.float32),
            lsem0=pltpu.SemaphoreType.DMA,
            lsem1=pltpu.SemaphoreType.DMA,
            gsem0=pltpu.SemaphoreType.DMA,
            gsem1=pltpu.SemaphoreType.DMA,
            gsem2=pltpu.SemaphoreType.DMA,
            gsem3=pltpu.SemaphoreType.DMA,
            ssem0=pltpu.SemaphoreType.DMA,
            ssem1=pltpu.SemaphoreType.DMA,
            ssem2=pltpu.SemaphoreType.DMA,
            ssem3=pltpu.SemaphoreType.DMA,
            gsem4=pltpu.SemaphoreType.DMA,
            gsem5=pltpu.SemaphoreType.DMA,
            gsem6=pltpu.SemaphoreType.DMA,
            gsem7=pltpu.SemaphoreType.DMA,
            ssem4=pltpu.SemaphoreType.DMA,
            ssem5=pltpu.SemaphoreType.DMA,
            ssem6=pltpu.SemaphoreType.DMA,
            ssem7=pltpu.SemaphoreType.DMA,
        ),
        compiler_params=pltpu.CompilerParams(use_tc_tiling_on_sc=False),
    )
    def sc_call(x16, rows_h, cols_h, vals_h, out_h, acc, xsh, cidx, ridx, vals,
                g0, g1, g2, g3, g4, g5, g6, g7, lsem0, lsem1,
                gsem0, gsem1, gsem2, gsem3, ssem0, ssem1, ssem2, ssem3,
                gsem4, gsem5, gsem6, gsem7, ssem4, ssem5, ssem6, ssem7):
        c = lax.axis_index("c")
        s = lax.axis_index("s")

        z16 = jnp.zeros((LANES,), jnp.float32)

        def zgroup(i, _):
            base = i * 16
            for u in range(16):
                g0[base + u] = z16
            return 0

        lax.fori_loop(0, CHUNK // 16, zgroup, 0)
        for i in range(rows_per_tile // CHUNK):
            pltpu.sync_copy(
                g0, acc.at[pl.ds(s * rows_per_tile + i * CHUNK, CHUNK)]
            )
        xsl = pl.ds(s * rows_per_tile, rows_per_tile)
        pltpu.sync_copy(x16.at[xsl], xsh.at[xsl])
        plsc.subcore_barrier()

        tile_id = s * NUM_CORES + c
        cbase = tile_id * (per_tile // CHUNK)

        gbufs = (g0, g1, g2, g3, g4, g5, g6, g7)
        gsems = (gsem0, gsem1, gsem2, gsem3, gsem4, gsem5, gsem6, gsem7)
        ssems = (ssem0, ssem1, ssem2, ssem3, ssem4, ssem5, ssem6, ssem7)
        lsems = (lsem0, lsem1)

        def issue_linear(p, t):
            coff = cbase + t * SUBCHUNKS
            pltpu.async_copy(
                rows_h.at[pl.ds(coff, SUBCHUNKS)], ridx.at[p], lsems[p]
            )
            pltpu.async_copy(
                cols_h.at[pl.ds(coff, SUBCHUNKS)], cidx.at[p], lsems[p]
            )
            pltpu.async_copy(
                vals_h.at[pl.ds(coff * CHUNK, STAGE)],
                vals.at[pl.ds(p * STAGE, STAGE)],
                lsems[p],
            )

        def wait_linear(p):
            pltpu.make_async_copy(
                rows_h.at[pl.ds(0, SUBCHUNKS)], ridx.at[p], lsems[p]
            ).wait()
            pltpu.make_async_copy(
                cols_h.at[pl.ds(0, SUBCHUNKS)], cidx.at[p], lsems[p]
            ).wait()
            pltpu.make_async_copy(
                vals_h.at[pl.ds(0, STAGE)],
                vals.at[pl.ds(p * STAGE, STAGE)],
                lsems[p],
            ).wait()

        def compute_inplace(gb, voff):
            def group(i16, _):
                vv = vals[pl.ds(voff + i16 * 16, 16)]
                base = i16 * 16
                for u in range(16):
                    gb[base + u] = gb[base + u] * vv[u]
                return 0

            lax.fori_loop(0, CHUNK // 16, group, 0)

        def process_stage(p):
            ci = cidx.at[p]
            ri = ridx.at[p]
            gd = [
                pltpu.async_copy(xsh.at[ci.at[j]], gbufs[j], gsems[j])
                for j in range(SUBCHUNKS)
            ]
            sd = []
            for j in range(SUBCHUNKS):
                gd[j].wait()
                compute_inplace(gbufs[j], p * STAGE + j * CHUNK)
                sd.append(
                    pltpu.async_copy(
                        gbufs[j], acc.at[ri.at[j]], ssems[j], add=True
                    )
                )
            for d in sd:
                d.wait()

        issue_linear(0, 0)

        def pair_body(t2, _):
            t0 = 2 * t2
            wait_linear(0)
            issue_linear(1, t0 + 1)
            process_stage(0)
            wait_linear(1)

            @pl.when(t2 + 1 < n_stage // 2)
            def _():
                issue_linear(0, t0 + 2)

            process_stage(1)
            return 0

        lax.fori_loop(0, n_stage // 2, pair_body, 0)

        plsc.subcore_barrier()
        def out_body(i, _):
            sl = pl.ds(s * rows_per_tile + i * CHUNK, CHUNK)
            pltpu.sync_copy(acc.at[sl], g0)
            pltpu.sync_copy(g0, out_h.at[c].at[sl])
            return 0

        lax.fori_loop(0, rows_per_tile // CHUNK, out_body, 0)

    return sc_call


def _combine(partials):
    two, n, lanes = partials.shape

    def body(p_ref, o_ref):
        o_ref[...] = p_ref[0] + p_ref[1]

    return pl.pallas_call(
        body,
        out_shape=jax.ShapeDtypeStruct((n, lanes), jnp.float32),
    )(partials)


def kernel(x_batched, rows, cols, values):
    B, N = x_batched.shape
    nnz = rows.shape[0]

    unit = NUM_TILES * STAGE * 2
    nnz_pad = ((nnz + unit - 1) // unit) * unit
    pad = nnz_pad - nnz
    if pad:
        spread = (jnp.arange(pad, dtype=jnp.int32) * 97) % N
        rows = jnp.concatenate([rows, spread])
        cols = jnp.concatenate([cols, spread])
        values = jnp.concatenate([values, jnp.zeros((pad,), values.dtype)])

    x16 = jnp.concatenate(
        [x_batched.T, jnp.zeros((N, LANES - B), jnp.float32)], axis=1
    )

    rows2d = rows.reshape(-1, CHUNK)
    cols2d = cols.reshape(-1, CHUNK)
    out2 = _make_sc_call(N, nnz_pad)(x16, rows2d, cols2d, values)
    out = _combine(out2)
    return out[:, :B].T


# speedup vs baseline: 1.6318x; 1.0010x over previous
import functools

import jax
import jax.numpy as jnp
from jax import lax
from jax.experimental import pallas as pl
from jax.experimental.pallas import tpu as pltpu
from jax.experimental.pallas import tpu_sc as plsc

NUM_CORES = 2
NUM_SUBCORES = 16
NUM_TILES = NUM_CORES * NUM_SUBCORES
CHUNK = 512
SUBCHUNKS = 4
STAGE = CHUNK * SUBCHUNKS
LANES = 16


def _make_sc_call(N: int, nnz_pad: int):
    per_tile = nnz_pad // NUM_TILES
    n_stage = per_tile // STAGE
    assert n_stage % 2 == 0
    rows_per_tile = N // NUM_SUBCORES

    mesh = plsc.VectorSubcoreMesh(
        core_axis_name="c", subcore_axis_name="s", num_cores=NUM_CORES
    )

    @functools.partial(
        pl.kernel,
        out_type=jax.ShapeDtypeStruct((NUM_CORES, N, LANES), jnp.float32),
        mesh=mesh,
        scratch_types=dict(
            acc=pltpu.MemorySpace.VMEM_SHARED((N, LANES), jnp.float32),
            xsh=pltpu.MemorySpace.VMEM_SHARED((N, LANES), jnp.float32),
            cidx=pltpu.MemorySpace.VMEM((2, SUBCHUNKS, CHUNK), jnp.int32),
            ridx=pltpu.MemorySpace.VMEM((2, SUBCHUNKS, CHUNK), jnp.int32),
            vals=pltpu.MemorySpace.VMEM((2 * STAGE,), jnp.float32),
            g0=pltpu.MemorySpace.VMEM((CHUNK, LANES), jnp.float32),
            g1=pltpu.MemorySpace.VMEM((CHUNK, LANES), jnp.float32),
            g2=pltpu.MemorySpace.VMEM((CHUNK, LANES), jnp.float32),
            g3=pltpu.MemorySpace.VMEM((CHUNK, LANES), jnp.float32),
            g4=pltpu.MemorySpace.VMEM((CHUNK, LANES), jnp.float32),
            g5=pltpu.MemorySpace.VMEM((CHUNK, LANES), jnp.float32),
            g6=pltpu.MemorySpace.VMEM((CHUNK, LANES), jnp.float32),
            g7=pltpu.MemorySpace.VMEM((CHUNK, LANES), jnp.float32),
            lsem0=pltpu.SemaphoreType.DMA,
            lsem1=pltpu.SemaphoreType.DMA,
            gsem0=pltpu.SemaphoreType.DMA,
            gsem1=pltpu.SemaphoreType.DMA,
            gsem2=pltpu.SemaphoreType.DMA,
            gsem3=pltpu.SemaphoreType.DMA,
            ssem0=pltpu.SemaphoreType.DMA,
            ssem1=pltpu.SemaphoreType.DMA,
            ssem2=pltpu.SemaphoreType.DMA,
            ssem3=pltpu.SemaphoreType.DMA,
            gsem4=pltpu.SemaphoreType.DMA,
            gsem5=pltpu.SemaphoreType.DMA,
            gsem6=pltpu.SemaphoreType.DMA,
            gsem7=pltpu.SemaphoreType.DMA,
            ssem4=pltpu.SemaphoreType.DMA,
            ssem5=pltpu.SemaphoreType.DMA,
            ssem6=pltpu.SemaphoreType.DMA,
            ssem7=pltpu.SemaphoreType.DMA,
        ),
        compiler_params=pltpu.CompilerParams(use_tc_tiling_on_sc=False),
    )
    def sc_call(x16, rows_h, cols_h, vals_h, out_h, acc, xsh, cidx, ridx, vals,
                g0, g1, g2, g3, g4, g5, g6, g7, lsem0, lsem1,
                gsem0, gsem1, gsem2, gsem3, ssem0, ssem1, ssem2, ssem3,
                gsem4, gsem5, gsem6, gsem7, ssem4, ssem5, ssem6, ssem7):
        c = lax.axis_index("c")
        s = lax.axis_index("s")

        z16 = jnp.zeros((LANES,), jnp.float32)

        def zgroup(i, _):
            base = i * 16
            for u in range(16):
                g0[base + u] = z16
            return 0

        lax.fori_loop(0, CHUNK // 16, zgroup, 0)
        for i in range(rows_per_tile // CHUNK):
            pltpu.sync_copy(
                g0, acc.at[pl.ds(s * rows_per_tile + i * CHUNK, CHUNK)]
            )
        xsl = pl.ds(s * rows_per_tile, rows_per_tile)
        pltpu.sync_copy(x16.at[xsl], xsh.at[xsl])
        plsc.subcore_barrier()

        tile_id = s * NUM_CORES + c
        cbase = tile_id * (per_tile // CHUNK)

        gbufs = (g0, g1, g2, g3, g4, g5, g6, g7)
        gsems = (gsem0, gsem1, gsem2, gsem3, gsem4, gsem5, gsem6, gsem7)
        ssems = (ssem0, ssem1, ssem2, ssem3, ssem4, ssem5, ssem6, ssem7)
        lsems = (lsem0, lsem1)

        def issue_linear(p, t):
            coff = cbase + t * SUBCHUNKS
            pltpu.async_copy(
                rows_h.at[pl.ds(coff, SUBCHUNKS)], ridx.at[p], lsems[p]
            )
            pltpu.async_copy(
                cols_h.at[pl.ds(coff, SUBCHUNKS)], cidx.at[p], lsems[p]
            )
            pltpu.async_copy(
                vals_h.at[pl.ds(coff * CHUNK, STAGE)],
                vals.at[pl.ds(p * STAGE, STAGE)],
                lsems[p],
            )

        def wait_linear(p):
            pltpu.make_async_copy(
                rows_h.at[pl.ds(0, SUBCHUNKS)], ridx.at[p], lsems[p]
            ).wait()
            pltpu.make_async_copy(
                cols_h.at[pl.ds(0, SUBCHUNKS)], cidx.at[p], lsems[p]
            ).wait()
            pltpu.make_async_copy(
                vals_h.at[pl.ds(0, STAGE)],
                vals.at[pl.ds(p * STAGE, STAGE)],
                lsems[p],
            ).wait()

        def compute_inplace(gb, voff):
            def group(i16, _):
                vv = vals[pl.ds(voff + i16 * 16, 16)]
                base = i16 * 16
                for u in range(16):
                    gb[base + u] = gb[base + u] * vv[u]
                return 0

            lax.fori_loop(0, CHUNK // 16, group, 0)

        def process_stage(p):
            ci = cidx.at[p]
            ri = ridx.at[p]
            gd = [
                pltpu.async_copy(xsh.at[ci.at[j]], gbufs[j], gsems[j])
                for j in range(SUBCHUNKS)
            ]
            sd = []
            for j in range(SUBCHUNKS):
                gd[j].wait()
                compute_inplace(gbufs[j], p * STAGE + j * CHUNK)
                sd.append(
                    pltpu.async_copy(
                        gbufs[j], acc.at[ri.at[j]], ssems[j], add=True
                    )
                )
            for d in sd:
                d.wait()

        issue_linear(0, 0)

        def pair_body(t2, _):
            t0 = 2 * t2
            wait_linear(0)
            issue_linear(1, t0 + 1)
            process_stage(0)
            wait_linear(1)

            @pl.when(t2 + 1 < n_stage // 2)
            def _():
                issue_linear(0, t0 + 2)

            process_stage(1)
            return 0

        lax.fori_loop(0, n_stage // 2, pair_body, 0)

        plsc.subcore_barrier()
        sl = pl.ds(s * rows_per_tile, rows_per_tile)
        pltpu.sync_copy(acc.at[sl], out_h.at[c].at[sl])

    return sc_call


def _combine(partials):
    two, n, lanes = partials.shape

    def body(p_ref, o_ref):
        o_ref[...] = p_ref[0] + p_ref[1]

    return pl.pallas_call(
        body,
        out_shape=jax.ShapeDtypeStruct((n, lanes), jnp.float32),
    )(partials)


def kernel(x_batched, rows, cols, values):
    B, N = x_batched.shape
    nnz = rows.shape[0]

    unit = NUM_TILES * STAGE * 2
    nnz_pad = ((nnz + unit - 1) // unit) * unit
    pad = nnz_pad - nnz
    if pad:
        spread = (jnp.arange(pad, dtype=jnp.int32) * 97) % N
        rows = jnp.concatenate([rows, spread])
        cols = jnp.concatenate([cols, spread])
        values = jnp.concatenate([values, jnp.zeros((pad,), values.dtype)])

    x16 = jnp.concatenate(
        [x_batched.T, jnp.zeros((N, LANES - B), jnp.float32)], axis=1
    )

    rows2d = rows.reshape(-1, CHUNK)
    cols2d = cols.reshape(-1, CHUNK)
    out2 = _make_sc_call(N, nnz_pad)(x16, rows2d, cols2d, values)
    out = _combine(out2)
    return out[:, :B].T
